# Initial kernel scaffold; baseline (speedup 1.0000x reference)
import jax
import jax.numpy as jnp
from jax.experimental import pallas as pl


def kernel(hidden_states, K1_w, K1_b, V1_w, V1_b, ReadingHead, bidirection_weight):
    raise NotImplementedError("write your pallas kernel here")


# speedup vs baseline: 16.7931x; 16.7931x over previous
import functools

import jax
import jax.numpy as jnp
from jax import lax
from jax.experimental import pallas as pl
from jax.experimental.pallas import tpu as pltpu
from jax.experimental.pallas import tpu_sc as plsc

BS = 4
L = 512
HIDDEN = 2048
NH = 32
HD = 64
R2 = 8
LANES = BS * NH
TM = 256
TN = 512
NJ = HIDDEN // TN
NI = (BS * L) // TM
HPT = TN // HD
BIG = 1 << 20


def _mm_kernel(a_ref, k1w_ref, v1w_ref, k1b_ref, v1b_ref, w2_ref,
               sc_ref, v1_ref):
    a = a_ref[...]
    dn = (((1,), (1,)), ((), ()))
    k1 = lax.dot_general(a, k1w_ref[...], dn,
                         precision=lax.Precision.HIGHEST,
                         preferred_element_type=jnp.float32)
    k1 = jnp.maximum(k1 + k1b_ref[...], 0.0)
    sc = lax.dot_general(k1, w2_ref[0], (((1,), (0,)), ((), ())),
                         precision=lax.Precision.HIGHEST,
                         preferred_element_type=jnp.float32)
    sc_ref[...] = sc[None]
    v1 = lax.dot_general(a, v1w_ref[...], dn,
                         precision=lax.Precision.HIGHEST,
                         preferred_element_type=jnp.float32)
    v1_ref[...] = jnp.maximum(v1 + v1b_ref[...], 0.0)


def _matmuls(a, k1w, v1w, k1b, v1b, w2):
    grid = (NJ, NI)
    return pl.pallas_call(
        _mm_kernel,
        grid=grid,
        in_specs=[
            pl.BlockSpec((TM, HIDDEN), lambda j, i: (i, 0)),
            pl.BlockSpec((TN, HIDDEN), lambda j, i: (j, 0)),
            pl.BlockSpec((TN, HIDDEN), lambda j, i: (j, 0)),
            pl.BlockSpec((1, TN), lambda j, i: (0, j)),
            pl.BlockSpec((1, TN), lambda j, i: (0, j)),
            pl.BlockSpec((1, TN, HPT), lambda j, i: (j, 0, 0)),
        ],
        out_specs=[
            pl.BlockSpec((1, TM, HPT), lambda j, i: (j, i, 0)),
            pl.BlockSpec((TM, TN), lambda j, i: (i, j)),
        ],
        out_shape=[
            jax.ShapeDtypeStruct((NJ, BS * L, HPT), jnp.float32),
            jax.ShapeDtypeStruct((BS * L, HIDDEN), jnp.float32),
        ],
    )(a, k1w, v1w, k1b, v1b, w2)


def _make_sc_kernel():
    mesh = plsc.VectorSubcoreMesh(core_axis_name="c", subcore_axis_name="s")
    NV = L // 16

    @functools.partial(
        pl.kernel,
        mesh=mesh,
        out_type=jax.ShapeDtypeStruct((LANES, L, HD), jnp.float32),
        compiler_params=pltpu.CompilerParams(
            needs_layout_passes=False, use_tc_tiling_on_sc=False),
        scratch_types=[
            pltpu.VMEM((L,), jnp.float32),
            pltpu.VMEM((L,), jnp.float32),
            pltpu.VMEM((L + 16,), jnp.int32),
            pltpu.VMEM((L,), jnp.int32),
            pltpu.VMEM((L,), jnp.int32),
            pltpu.VMEM((L,), jnp.int32),
            pltpu.VMEM((L,), jnp.int32),
            pltpu.VMEM((NV, 128), jnp.int32),
            pltpu.VMEM((128, HD), jnp.float32),
            pltpu.VMEM((L, HD), jnp.float32),
            pltpu.VMEM((R2, 16), jnp.float32),
            pltpu.VMEM((16,), jnp.int32),
            pltpu.VMEM((16,), jnp.float32),
            pltpu.SemaphoreType.DMA,
        ],
    )
    def sck(scores_hbm, v1_hbm, wbc_hbm, out_hbm,
            x_loc, e_loc, ev_loc, cnt_loc, c0_loc, c1_loc, P_loc,
            idx2d, rows, out_loc, w_loc, tmp16, tmpf16, sem):
        wid = lax.axis_index("s") * 2 + lax.axis_index("c")
        iota = lax.iota(jnp.int32, 16)
        thr = jnp.float32(1.5 / L)

        def fsum_all(vec):
            for k in (1, 2, 4, 8):
                tmpf16[...] = vec
                vec = vec + plsc.load_gather(tmpf16, [lax.bitwise_xor(iota, k)])
            return vec

        def fmax_all(vec):
            for k in (1, 2, 4, 8):
                tmpf16[...] = vec
                vec = jnp.maximum(vec, plsc.load_gather(tmpf16, [lax.bitwise_xor(iota, k)]))
            return vec

        def ibcast(vec, lane):
            tmp16[...] = vec
            return plsc.load_gather(tmp16, [jnp.full((16,), lane, jnp.int32)])

        def task(kk, _):
            bh = wid * 4 + kk
            b = bh // NH
            h = bh - b * NH
            base = b * (L * NH) + h
            pltpu.sync_copy(scores_hbm.at[bh], x_loc)
            pltpu.sync_copy(wbc_hbm.at[h], w_loc)


            def p1(v, m):
                xv = x_loc[pl.ds(v * 16, 16)]
                return jnp.maximum(m, xv)
            mlane = lax.fori_loop(0, NV, p1, jnp.full((16,), -3.4e38, jnp.float32))
            Mv = fmax_all(mlane)
            expneg = jnp.exp(-Mv)

            def p2(v, s):
                xv = x_loc[pl.ds(v * 16, 16)]
                evreg = jnp.exp(xv - Mv)
                e_loc[pl.ds(v * 16, 16)] = evreg
                return s + evreg
            slane = lax.fori_loop(0, NV, p2, jnp.zeros((16,), jnp.float32))
            Sinv = 1.0 / fsum_all(slane)

            def p3(v, carry):
                cnt_c, m_c = carry
                t_vec = iota + v * 16
                p = e_loc[pl.ds(v * 16, 16)] * Sinv + expneg
                evb = p > thr
                ev_loc[pl.ds(v * 16, 16)] = evb.astype(jnp.int32)
                push = jnp.logical_and(evb, t_vec > 0)
                pushi = push.astype(jnp.int32)
                cntv = plsc.cumsum(pushi) + cnt_c
                cnt_loc[pl.ds(v * 16, 16)] = cntv
                mval = jnp.where(evb, t_vec, -1)
                mm = jnp.maximum(plsc.cummax(mval), m_c)
                c0_loc[pl.ds(v * 16, 16)] = jnp.where(mm < 0, t_vec, mm)
                plsc.store_scatter(P_loc, [cntv], t_vec, mask=push)
                return (ibcast(cntv, 15), ibcast(mm, 15))
            lax.fori_loop(0, NV, p3,
                          (jnp.zeros((16,), jnp.int32),
                           jnp.full((16,), -1, jnp.int32)))
            ev_loc[pl.ds(L, 16)] = ev_loc[pl.ds(0, 16)]

            def p4(vv, minc):
                v = NV - 1 - vv
                t_vec = iota + v * 16
                e2 = ev_loc[pl.ds(v * 16 + 1, 16)]
                val = jnp.where(e2 != 0, t_vec + 1, BIG)
                sfx = -lax.rev(plsc.cummax(lax.rev(-val, (0,))), (0,))
                run = jnp.minimum(sfx, minc)
                c1v = jnp.where(run >= BIG, t_vec + 1, run)
                c1v = jnp.where(t_vec == 0, 0, c1v)
                c1_loc[pl.ds(v * 16, 16)] = c1v
                return ibcast(run, 0)
            lax.fori_loop(0, NV, p4, jnp.full((16,), BIG, jnp.int32))

            def p5(v, _):
                t_vec = iota + v * 16
                cntv = cnt_loc[pl.ds(v * 16, 16)]
                for k in range(1, 5):
                    j = cntv - (k - 1)
                    valid = j >= 1
                    jc = jnp.minimum(jnp.maximum(j, 0), L - 1)
                    s = plsc.load_gather(P_loc, [jc])
                    sc1 = jnp.minimum(jnp.maximum(s - 1, 0), L - 1)
                    p0 = plsc.load_gather(c0_loc, [sc1])
                    p1v = plsc.load_gather(c1_loc, [sc1])
                    for rr, pv in ((2 * (k - 1), p0), (2 * k - 1, p1v)):
                        pv = jnp.where(valid, pv, 0)
                        pv = jnp.minimum(pv, L - 1)
                        q = t_vec * 8 + rr
                        row = lax.shift_right_logical(q, 7)
                        col = lax.bitwise_and(q, 127)
                        plsc.store_scatter(idx2d, [row, col], pv * NH + base)
                return 0
            lax.fori_loop(0, NV, p5, 0)

            wv = [w_loc[r] for r in range(R2)]

            def p6(g, _):
                pltpu.async_copy(v1_hbm.at[idx2d.at[g]], rows, sem).wait()

                def acc_t(lt, _2):
                    t = g * 16 + lt
                    for d in range(4):
                        a = wv[0] * rows[lt * 8, pl.ds(d * 16, 16)]
                        for r in range(1, R2):
                            a = a + wv[r] * rows[lt * 8 + r, pl.ds(d * 16, 16)]
                        out_loc[t, pl.ds(d * 16, 16)] = a
                    return 0
                lax.fori_loop(0, 16, acc_t, 0)
                return 0
            lax.fori_loop(0, NV, p6, 0)

            pltpu.sync_copy(out_loc, out_hbm.at[bh])
            return 0

        lax.fori_loop(0, 4, task, 0)

    return sck


@functools.lru_cache(maxsize=1)
def _get_sck():
    return _make_sc_kernel()


def kernel(hidden_states, K1_w, K1_b, V1_w, V1_b, ReadingHead, bidirection_weight):
    bs, length, hidden = hidden_states.shape
    a = hidden_states.reshape(bs * length, hidden)
    rh = ReadingHead.reshape(NJ, HPT, HD)
    eye = jnp.eye(HPT, dtype=jnp.float32)
    w2 = jnp.einsum('jhd,he->jhde', rh, eye).reshape(NJ, TN, HPT)
    s3, v1 = _matmuls(a, K1_w, V1_w, K1_b.reshape(1, -1), V1_b.reshape(1, -1), w2)
    scores_bh = s3.reshape(NJ, bs, length, HPT).transpose(1, 0, 3, 2).reshape(LANES, length)
    v1flat = v1.reshape(bs * length * NH, HD)
    wbc = jnp.broadcast_to(
        bidirection_weight.reshape(NH, R2, 1).astype(jnp.float32), (NH, R2, 16))
    out_bh = _get_sck()(scores_bh, v1flat, wbc)
    return out_bh.reshape(bs, NH, length, HD).transpose(0, 2, 1, 3)


# speedup vs baseline: 31.5148x; 1.8767x over previous
import functools

import jax
import jax.numpy as jnp
from jax import lax
from jax.experimental import pallas as pl
from jax.experimental.pallas import tpu as pltpu
from jax.experimental.pallas import tpu_sc as plsc

BS = 4
L = 512
HIDDEN = 2048
NH = 32
HD = 64
R2 = 8
LANES = BS * NH
TM = 256
TN = 512
NJ = HIDDEN // TN
NI = (BS * L) // TM
HPT = TN // HD
BIG = 1 << 20


def _mm_kernel(a_ref, k1w_ref, v1w_ref, k1b_ref, v1b_ref, w2_ref,
               sc_ref, v1_ref):
    a = a_ref[...]
    dn = (((1,), (1,)), ((), ()))
    k1 = lax.dot_general(a, k1w_ref[...], dn,
                         preferred_element_type=jnp.float32)
    k1 = jnp.maximum(k1 + k1b_ref[...], 0.0)
    sc = lax.dot_general(k1, w2_ref[0], (((1,), (0,)), ((), ())),
                         preferred_element_type=jnp.float32)
    sc_ref[...] = sc[None]
    v1 = lax.dot_general(a, v1w_ref[...], dn,
                         preferred_element_type=jnp.float32)
    v1_ref[...] = jnp.maximum(v1 + v1b_ref[...], 0.0)


def _matmuls(a, k1w, v1w, k1b, v1b, w2):
    grid = (NJ, NI)
    return pl.pallas_call(
        _mm_kernel,
        grid=grid,
        in_specs=[
            pl.BlockSpec((TM, HIDDEN), lambda j, i: (i, 0)),
            pl.BlockSpec((TN, HIDDEN), lambda j, i: (j, 0)),
            pl.BlockSpec((TN, HIDDEN), lambda j, i: (j, 0)),
            pl.BlockSpec((1, TN), lambda j, i: (0, j)),
            pl.BlockSpec((1, TN), lambda j, i: (0, j)),
            pl.BlockSpec((1, TN, HPT), lambda j, i: (j, 0, 0)),
        ],
        out_specs=[
            pl.BlockSpec((1, TM, HPT), lambda j, i: (j, i, 0)),
            pl.BlockSpec((TM, TN), lambda j, i: (i, j)),
        ],
        out_shape=[
            jax.ShapeDtypeStruct((NJ, BS * L, HPT), jnp.float32),
            jax.ShapeDtypeStruct((BS * L, HIDDEN), jnp.float32),
        ],
    )(a, k1w, v1w, k1b, v1b, w2)


def _make_sc_kernel():
    mesh = plsc.VectorSubcoreMesh(core_axis_name="c", subcore_axis_name="s")
    NV = L // 16

    @functools.partial(
        pl.kernel,
        mesh=mesh,
        out_type=jax.ShapeDtypeStruct((LANES, L, HD), jnp.float32),
        compiler_params=pltpu.CompilerParams(
            needs_layout_passes=False, use_tc_tiling_on_sc=False),
        scratch_types=[
            pltpu.VMEM((L,), jnp.float32),
            pltpu.VMEM((L,), jnp.float32),
            pltpu.VMEM((L + 16,), jnp.int32),
            pltpu.VMEM((L,), jnp.int32),
            pltpu.VMEM((L,), jnp.int32),
            pltpu.VMEM((L,), jnp.int32),
            pltpu.VMEM((L,), jnp.int32),
            pltpu.VMEM((NV, 128), jnp.int32),
            pltpu.VMEM((2, 128, HD), jnp.float32),
            pltpu.VMEM((L, HD), jnp.float32),
            pltpu.VMEM((R2, 16), jnp.float32),
            pltpu.VMEM((16,), jnp.int32),
            pltpu.VMEM((16,), jnp.float32),
            pltpu.SemaphoreType.DMA((2,)),
        ],
    )
    def sck(scores_hbm, v1_hbm, wbc_hbm, out_hbm,
            x_loc, e_loc, ev_loc, cnt_loc, c0_loc, c1_loc, P_loc,
            idx2d, rows, out_loc, w_loc, tmp16, tmpf16, sem):
        wid = lax.axis_index("s") * 2 + lax.axis_index("c")
        iota = lax.iota(jnp.int32, 16)
        thr = jnp.float32(1.5 / L)

        def fsum_all(vec):
            for k in (1, 2, 4, 8):
                tmpf16[...] = vec
                vec = vec + plsc.load_gather(tmpf16, [lax.bitwise_xor(iota, k)])
            return vec

        def fmax_all(vec):
            for k in (1, 2, 4, 8):
                tmpf16[...] = vec
                vec = jnp.maximum(vec, plsc.load_gather(tmpf16, [lax.bitwise_xor(iota, k)]))
            return vec

        def ibcast(vec, lane):
            tmp16[...] = vec
            return plsc.load_gather(tmp16, [jnp.full((16,), lane, jnp.int32)])

        def task(kk, _):
            bh = wid * 4 + kk
            b = bh // NH
            h = bh - b * NH
            base = b * (L * NH) + h
            pltpu.sync_copy(scores_hbm.at[bh], x_loc)
            pltpu.sync_copy(wbc_hbm.at[h], w_loc)


            def p1(v, m):
                xv = x_loc[pl.ds(v * 16, 16)]
                return jnp.maximum(m, xv)
            mlane = lax.fori_loop(0, NV, p1, jnp.full((16,), -3.4e38, jnp.float32))
            Mv = fmax_all(mlane)
            expneg = jnp.exp(-Mv)

            def p2(v, s):
                xv = x_loc[pl.ds(v * 16, 16)]
                evreg = jnp.exp(xv - Mv)
                e_loc[pl.ds(v * 16, 16)] = evreg
                return s + evreg
            slane = lax.fori_loop(0, NV, p2, jnp.zeros((16,), jnp.float32))
            Sinv = 1.0 / fsum_all(slane)

            def p3(v, carry):
                cnt_c, m_c = carry
                t_vec = iota + v * 16
                p = e_loc[pl.ds(v * 16, 16)] * Sinv + expneg
                evb = p > thr
                ev_loc[pl.ds(v * 16, 16)] = evb.astype(jnp.int32)
                push = jnp.logical_and(evb, t_vec > 0)
                pushi = push.astype(jnp.int32)
                cntv = plsc.cumsum(pushi) + cnt_c
                cnt_loc[pl.ds(v * 16, 16)] = cntv
                mval = jnp.where(evb, t_vec, -1)
                mm = jnp.maximum(plsc.cummax(mval), m_c)
                c0_loc[pl.ds(v * 16, 16)] = jnp.where(mm < 0, t_vec, mm)
                plsc.store_scatter(P_loc, [cntv], t_vec, mask=push)
                return (ibcast(cntv, 15), ibcast(mm, 15))
            lax.fori_loop(0, NV, p3,
                          (jnp.zeros((16,), jnp.int32),
                           jnp.full((16,), -1, jnp.int32)))
            ev_loc[pl.ds(L, 16)] = ev_loc[pl.ds(0, 16)]

            def p4(vv, minc):
                v = NV - 1 - vv
                t_vec = iota + v * 16
                e2 = ev_loc[pl.ds(v * 16 + 1, 16)]
                val = jnp.where(e2 != 0, t_vec + 1, BIG)
                sfx = -lax.rev(plsc.cummax(lax.rev(-val, (0,))), (0,))
                run = jnp.minimum(sfx, minc)
                c1v = jnp.where(run >= BIG, t_vec + 1, run)
                c1v = jnp.where(t_vec == 0, 0, c1v)
                c1_loc[pl.ds(v * 16, 16)] = c1v
                return ibcast(run, 0)
            lax.fori_loop(0, NV, p4, jnp.full((16,), BIG, jnp.int32))

            def p5(v, _):
                t_vec = iota + v * 16
                cntv = cnt_loc[pl.ds(v * 16, 16)]
                for k in range(1, 5):
                    j = cntv - (k - 1)
                    valid = j >= 1
                    jc = jnp.minimum(jnp.maximum(j, 0), L - 1)
                    s = plsc.load_gather(P_loc, [jc])
                    sc1 = jnp.minimum(jnp.maximum(s - 1, 0), L - 1)
                    p0 = plsc.load_gather(c0_loc, [sc1])
                    p1v = plsc.load_gather(c1_loc, [sc1])
                    for rr, pv in ((2 * (k - 1), p0), (2 * k - 1, p1v)):
                        pv = jnp.where(valid, pv, 0)
                        pv = jnp.minimum(pv, L - 1)
                        q = t_vec * 8 + rr
                        row = lax.shift_right_logical(q, 7)
                        col = lax.bitwise_and(q, 127)
                        plsc.store_scatter(idx2d, [row, col], pv * NH + base)
                return 0
            lax.fori_loop(0, NV, p5, 0)

            wv = [w_loc[r] for r in range(R2)]
            pltpu.async_copy(v1_hbm.at[idx2d.at[0]], rows.at[0], sem.at[0])

            def p6(g, _):
                slot = lax.rem(g, 2)
                nslot = lax.rem(g + 1, 2)

                @pl.when(g + 1 < NV)
                def _start_next():
                    pltpu.async_copy(v1_hbm.at[idx2d.at[g + 1]],
                                     rows.at[nslot], sem.at[nslot])
                pltpu.make_async_copy(v1_hbm.at[idx2d.at[g]],
                                      rows.at[slot], sem.at[slot]).wait()

                def acc_t(lt, _2):
                    t = g * 16 + lt
                    for d in range(4):
                        a = wv[0] * rows[slot, lt * 8, pl.ds(d * 16, 16)]
                        for r in range(1, R2):
                            a = a + wv[r] * rows[slot, lt * 8 + r, pl.ds(d * 16, 16)]
                        out_loc[t, pl.ds(d * 16, 16)] = a
                    return 0
                lax.fori_loop(0, 16, acc_t, 0)
                return 0
            lax.fori_loop(0, NV, p6, 0)

            pltpu.sync_copy(out_loc, out_hbm.at[bh])
            return 0

        lax.fori_loop(0, 4, task, 0)

    return sck


@functools.lru_cache(maxsize=1)
def _get_sck():
    return _make_sc_kernel()


def kernel(hidden_states, K1_w, K1_b, V1_w, V1_b, ReadingHead, bidirection_weight):
    bs, length, hidden = hidden_states.shape
    a = hidden_states.reshape(bs * length, hidden).astype(jnp.bfloat16)
    K1_w = K1_w.astype(jnp.bfloat16)
    V1_w = V1_w.astype(jnp.bfloat16)
    rh = ReadingHead.reshape(NJ, HPT, HD)
    eye = jnp.eye(HPT, dtype=jnp.float32)
    w2 = jnp.einsum('jhd,he->jhde', rh, eye).reshape(NJ, TN, HPT)
    s3, v1 = _matmuls(a, K1_w, V1_w, K1_b.reshape(1, -1), V1_b.reshape(1, -1), w2)
    scores_bh = s3.reshape(NJ, bs, length, HPT).transpose(1, 0, 3, 2).reshape(LANES, length)
    v1flat = v1.reshape(bs * length * NH, HD)
    wbc = jnp.broadcast_to(
        bidirection_weight.reshape(NH, R2, 1).astype(jnp.float32), (NH, R2, 16))
    out_bh = _get_sck()(scores_bh, v1flat, wbc)
    return out_bh.reshape(bs, NH, length, HD).transpose(0, 2, 1, 3)


# speedup vs baseline: 34.8223x; 1.1049x over previous
import functools

import jax
import jax.numpy as jnp
from jax import lax
from jax.experimental import pallas as pl
from jax.experimental.pallas import tpu as pltpu
from jax.experimental.pallas import tpu_sc as plsc

BS = 4
L = 512
HIDDEN = 2048
NH = 32
HD = 64
R2 = 8
LANES = BS * NH
TM = 512
TN = 512
NJ = HIDDEN // TN
NI = (BS * L) // TM
HPT = TN // HD
BIG = 1 << 20


def _mm_kernel(a_ref, k1w_ref, v1w_ref, k1b_ref, v1b_ref, w2_ref,
               sc_ref, v1_ref):
    a = a_ref[...]
    dn = (((1,), (1,)), ((), ()))
    k1 = lax.dot_general(a, k1w_ref[...], dn,
                         preferred_element_type=jnp.float32)
    k1 = jnp.maximum(k1 + k1b_ref[...], 0.0)
    sc = lax.dot_general(k1, w2_ref[0], (((1,), (0,)), ((), ())),
                         preferred_element_type=jnp.float32)
    sc_ref[...] = sc[None]
    v1 = lax.dot_general(a, v1w_ref[...], dn,
                         preferred_element_type=jnp.float32)
    v1_ref[...] = jnp.maximum(v1 + v1b_ref[...], 0.0)


def _matmuls(a, k1w, v1w, k1b, v1b, w2):
    grid = (NJ, NI)
    return pl.pallas_call(
        _mm_kernel,
        grid=grid,
        in_specs=[
            pl.BlockSpec((TM, HIDDEN), lambda j, i: (i, 0)),
            pl.BlockSpec((TN, HIDDEN), lambda j, i: (j, 0)),
            pl.BlockSpec((TN, HIDDEN), lambda j, i: (j, 0)),
            pl.BlockSpec((1, TN), lambda j, i: (0, j)),
            pl.BlockSpec((1, TN), lambda j, i: (0, j)),
            pl.BlockSpec((1, TN, HPT), lambda j, i: (j, 0, 0)),
        ],
        out_specs=[
            pl.BlockSpec((1, TM, HPT), lambda j, i: (j, i, 0)),
            pl.BlockSpec((TM, TN), lambda j, i: (i, j)),
        ],
        out_shape=[
            jax.ShapeDtypeStruct((NJ, BS * L, HPT), jnp.float32),
            jax.ShapeDtypeStruct((BS * L, HIDDEN), jnp.float32),
        ],
    )(a, k1w, v1w, k1b, v1b, w2)


def _make_sc_kernel():
    mesh = plsc.VectorSubcoreMesh(core_axis_name="c", subcore_axis_name="s")
    NV = L // 16

    @functools.partial(
        pl.kernel,
        mesh=mesh,
        out_type=jax.ShapeDtypeStruct((LANES, L, HD), jnp.float32),
        compiler_params=pltpu.CompilerParams(
            needs_layout_passes=False, use_tc_tiling_on_sc=False),
        scratch_types=[
            pltpu.VMEM((L,), jnp.float32),
            pltpu.VMEM((L,), jnp.float32),
            pltpu.VMEM((L + 16,), jnp.int32),
            pltpu.VMEM((L,), jnp.int32),
            pltpu.VMEM((L,), jnp.int32),
            pltpu.VMEM((L,), jnp.int32),
            pltpu.VMEM((L,), jnp.int32),
            pltpu.VMEM((L * R2,), jnp.int32),
            pltpu.VMEM((2, 512, HD), jnp.float32),
            pltpu.VMEM((L, HD), jnp.float32),
            pltpu.VMEM((R2, 16), jnp.float32),
            pltpu.VMEM((16,), jnp.int32),
            pltpu.VMEM((16,), jnp.float32),
            pltpu.SemaphoreType.DMA((2,)),
        ],
    )
    def sck(scores_hbm, v1_hbm, wbc_hbm, out_hbm,
            x_loc, e_loc, ev_loc, cnt_loc, c0_loc, c1_loc, P_loc,
            idx1d, rows, out_loc, w_loc, tmp16, tmpf16, sem):
        wid = lax.axis_index("s") * 2 + lax.axis_index("c")
        iota = lax.iota(jnp.int32, 16)
        thr = jnp.float32(1.5 / L)

        def fsum_all(vec):
            for k in (1, 2, 4, 8):
                tmpf16[...] = vec
                vec = vec + plsc.load_gather(tmpf16, [lax.bitwise_xor(iota, k)])
            return vec

        def fmax_all(vec):
            for k in (1, 2, 4, 8):
                tmpf16[...] = vec
                vec = jnp.maximum(vec, plsc.load_gather(tmpf16, [lax.bitwise_xor(iota, k)]))
            return vec

        def ibcast(vec, lane):
            tmp16[...] = vec
            return plsc.load_gather(tmp16, [jnp.full((16,), lane, jnp.int32)])

        def task(kk, _):
            bh = wid * 4 + kk
            b = bh // NH
            h = bh - b * NH
            base = b * (L * NH) + h
            pltpu.sync_copy(scores_hbm.at[bh], x_loc)
            pltpu.sync_copy(wbc_hbm.at[h], w_loc)


            def p1(v, m):
                xv = x_loc[pl.ds(v * 16, 16)]
                return jnp.maximum(m, xv)
            mlane = lax.fori_loop(0, NV, p1, jnp.full((16,), -3.4e38, jnp.float32))
            Mv = fmax_all(mlane)
            expneg = jnp.exp(-Mv)

            def p2(v, s):
                xv = x_loc[pl.ds(v * 16, 16)]
                evreg = jnp.exp(xv - Mv)
                e_loc[pl.ds(v * 16, 16)] = evreg
                return s + evreg
            slane = lax.fori_loop(0, NV, p2, jnp.zeros((16,), jnp.float32))
            Sinv = 1.0 / fsum_all(slane)

            def p3(v, carry):
                cnt_c, m_c = carry
                t_vec = iota + v * 16
                p = e_loc[pl.ds(v * 16, 16)] * Sinv + expneg
                evb = p > thr
                ev_loc[pl.ds(v * 16, 16)] = evb.astype(jnp.int32)
                push = jnp.logical_and(evb, t_vec > 0)
                pushi = push.astype(jnp.int32)
                cntv = plsc.cumsum(pushi) + cnt_c
                cnt_loc[pl.ds(v * 16, 16)] = cntv
                mval = jnp.where(evb, t_vec, -1)
                mm = jnp.maximum(plsc.cummax(mval), m_c)
                c0_loc[pl.ds(v * 16, 16)] = jnp.where(mm < 0, t_vec, mm)
                plsc.store_scatter(P_loc, [cntv], t_vec, mask=push)
                return (ibcast(cntv, 15), ibcast(mm, 15))
            lax.fori_loop(0, NV, p3,
                          (jnp.zeros((16,), jnp.int32),
                           jnp.full((16,), -1, jnp.int32)))
            ev_loc[pl.ds(L, 16)] = ev_loc[pl.ds(0, 16)]

            def p4(vv, minc):
                v = NV - 1 - vv
                t_vec = iota + v * 16
                e2 = ev_loc[pl.ds(v * 16 + 1, 16)]
                val = jnp.where(e2 != 0, t_vec + 1, BIG)
                sfx = -lax.rev(plsc.cummax(lax.rev(-val, (0,))), (0,))
                run = jnp.minimum(sfx, minc)
                c1v = jnp.where(run >= BIG, t_vec + 1, run)
                c1v = jnp.where(t_vec == 0, 0, c1v)
                c1_loc[pl.ds(v * 16, 16)] = c1v
                return ibcast(run, 0)
            lax.fori_loop(0, NV, p4, jnp.full((16,), BIG, jnp.int32))

            def p5(v, _):
                t_vec = iota + v * 16
                cntv = cnt_loc[pl.ds(v * 16, 16)]
                for k in range(1, 5):
                    j = cntv - (k - 1)
                    valid = j >= 1
                    jc = jnp.minimum(jnp.maximum(j, 0), L - 1)
                    s = plsc.load_gather(P_loc, [jc])
                    sc1 = jnp.minimum(jnp.maximum(s - 1, 0), L - 1)
                    p0 = plsc.load_gather(c0_loc, [sc1])
                    p1v = plsc.load_gather(c1_loc, [sc1])
                    for rr, pv in ((2 * (k - 1), p0), (2 * k - 1, p1v)):
                        pv = jnp.where(valid, pv, 0)
                        pv = jnp.minimum(pv, L - 1)
                        q = t_vec * 8 + rr
                        plsc.store_scatter(idx1d, [q], pv * NH + base)
                return 0
            lax.fori_loop(0, NV, p5, 0)

            NG = 8
            wv = [w_loc[r] for r in range(R2)]
            pltpu.async_copy(v1_hbm.at[idx1d.at[pl.ds(0, 512)]], rows.at[0], sem.at[0])

            def p6(g, _):
                slot = lax.rem(g, 2)
                nslot = lax.rem(g + 1, 2)

                @pl.when(g + 1 < NG)
                def _start_next():
                    pltpu.async_copy(v1_hbm.at[idx1d.at[pl.ds((g + 1) * 512, 512)]],
                                     rows.at[nslot], sem.at[nslot])
                pltpu.make_async_copy(v1_hbm.at[idx1d.at[pl.ds(g * 512, 512)]],
                                      rows.at[slot], sem.at[slot]).wait()

                def acc_t(lt, _2):
                    for c in range(4):
                        ltl = c * 16 + lt
                        t = g * 64 + ltl
                        for d in range(4):
                            a = wv[0] * rows[slot, ltl * 8, pl.ds(d * 16, 16)]
                            for r in range(1, R2):
                                a = a + wv[r] * rows[slot, ltl * 8 + r, pl.ds(d * 16, 16)]
                            out_loc[t, pl.ds(d * 16, 16)] = a
                    return 0
                lax.fori_loop(0, 16, acc_t, 0)
                return 0
            lax.fori_loop(0, NG, p6, 0)

            pltpu.sync_copy(out_loc, out_hbm.at[bh])
            return 0

        lax.fori_loop(0, 4, task, 0)

    return sck


@functools.lru_cache(maxsize=1)
def _get_sck():
    return _make_sc_kernel()


def kernel(hidden_states, K1_w, K1_b, V1_w, V1_b, ReadingHead, bidirection_weight):
    bs, length, hidden = hidden_states.shape
    a = hidden_states.reshape(bs * length, hidden).astype(jnp.bfloat16)
    K1_w = K1_w.astype(jnp.bfloat16)
    V1_w = V1_w.astype(jnp.bfloat16)
    rh = ReadingHead.reshape(NJ, HPT, HD)
    eye = jnp.eye(HPT, dtype=jnp.float32)
    w2 = jnp.einsum('jhd,he->jhde', rh, eye).reshape(NJ, TN, HPT)
    s3, v1 = _matmuls(a, K1_w, V1_w, K1_b.reshape(1, -1), V1_b.reshape(1, -1), w2)
    scores_bh = s3.reshape(NJ, bs, length, HPT).transpose(1, 0, 3, 2).reshape(LANES, length)
    v1flat = v1.reshape(bs * length * NH, HD)
    wbc = jnp.broadcast_to(
        bidirection_weight.reshape(NH, R2, 1).astype(jnp.float32), (NH, R2, 16))
    out_bh = _get_sck()(scores_bh, v1flat, wbc)
    return out_bh.reshape(bs, NH, length, HD).transpose(0, 2, 1, 3)
